# baseline (speedup 1.0000x reference)
import functools

import jax
import jax.numpy as jnp
from jax import lax
from jax.experimental import pallas as pl
from jax.experimental.pallas import tpu as pltpu
from jax.experimental.pallas import tpu_sc as plsc

_NC = 2
_NS = 16
_NW = _NC * _NS


_CHUNK = 128
_FIRE = 8


def _sc_gather_body(S_hbm, edge_hbm, out_hbm, S_sh, idx_v, out_v, sem, *, e_per_w, rows):
    sid = lax.axis_index("s")
    wid = sid * _NC + lax.axis_index("c")
    base = wid * e_per_w

    @pl.when(sid == 0)
    def _():
        pltpu.sync_copy(S_hbm, S_sh)

    zero = jnp.zeros((16,), jnp.int32)
    for t in range(e_per_w - e_per_w % 16, rows * _CHUNK, 16):
        idx_v[pl.ds(t, 16)] = zero
    cp_idx = pltpu.make_async_copy(
        edge_hbm.at[pl.ds(base, e_per_w)], idx_v.at[pl.ds(0, e_per_w)], sem)
    cp_idx.start()
    plsc.subcore_barrier()
    cp_idx.wait()

    def fire(j, _):
        pltpu.make_async_copy(
            S_sh.at[idx_v.at[pl.ds(j * _CHUNK, _CHUNK)]],
            out_v.at[pl.ds(j * _CHUNK, _CHUNK)],
            sem,
        ).start()
        return 0

    def drain(j, _):
        pltpu.make_async_copy(
            S_sh.at[idx_v.at[pl.ds(j * _CHUNK, _CHUNK)]],
            out_v.at[pl.ds(j * _CHUNK, _CHUNK)],
            sem,
        ).wait()
        return 0

    lax.fori_loop(0, rows, fire, 0)
    lax.fori_loop(0, rows, drain, 0)
    pltpu.sync_copy(out_v.at[pl.ds(0, e_per_w)],
                    out_hbm.at[pl.ds(base, e_per_w)])


def _neighbor_states(S_flat, edge_flat):
    e = edge_flat.shape[0]
    e_per_w = e // _NW
    rows = (e_per_w + _CHUNK - 1) // _CHUNK
    mesh = plsc.VectorSubcoreMesh(core_axis_name="c", subcore_axis_name="s")
    body = functools.partial(_sc_gather_body, e_per_w=e_per_w, rows=rows)
    return pl.kernel(
        body,
        mesh=mesh,
        out_type=jax.ShapeDtypeStruct((e,), jnp.int32),
        scratch_types=[
            pltpu.VMEM_SHARED((S_flat.shape[0],), jnp.int32),
            pltpu.VMEM((rows * _CHUNK,), jnp.int32),
            pltpu.VMEM((rows * _CHUNK,), jnp.int32),
            pltpu.SemaphoreType.DMA,
        ],
    )(S_flat, edge_flat)


def _tc_body(sj_ref, mij_ref, h_ref, mi_ref, S_ref, J_ref, U_ref, Ui_ref, M_ref, *, c, k, sg_n):
    ci = pl.program_id(0)
    sg = pl.program_id(1)
    s_per = c // sg_n

    @pl.when((ci == 0) & (sg == 0))
    def _():
        sj = sj_ref[...]
        mij = mij_ref[...]
        for s in range(c):
            M_ref[s] = (sj == s).astype(jnp.float32) * mij

    part = J_ref[0, 0] * M_ref[sg * s_per]
    for ds in range(1, s_per):
        part += J_ref[0, ds] * M_ref[sg * s_per + ds]
    ji = part.sum(axis=0, keepdims=True)
    sel = (S_ref[...] == ci).astype(jnp.float32)
    hm = h_ref[pl.ds(ci, 1), :] * mi_ref[...]

    @pl.when(sg == 0)
    def _():
        Ui_ref[pl.ds(ci, 1), :] = hm + ji

    @pl.when(sg != 0)
    def _():
        Ui_ref[pl.ds(ci, 1), :] += ji

    piece = jnp.where(sg == 0, ((hm + 0.5 * ji) * sel).sum(),
                      (0.5 * ji * sel).sum()).reshape(1, 1)

    @pl.when((ci == 0) & (sg == 0))
    def _():
        U_ref[...] = jnp.zeros((1, 1), jnp.float32)

    U_ref[...] += piece


def kernel(S, h, J, edge_idx, mask_i, mask_ij):
    B, N, K, C, _ = J.shape
    assert B == 1

    S_flat = S[0]
    edge_flat = jnp.transpose(edge_idx[0], (1, 0)).reshape(-1)
    sj = _neighbor_states(S_flat, edge_flat).reshape(K, N)

    Jt = jnp.transpose(J[0], (2, 3, 1, 0))
    h_cn = jnp.transpose(h[0], (1, 0))
    mij_kn = jnp.transpose(mask_ij[0], (1, 0))

    sg_n = 2
    body = functools.partial(_tc_body, c=C, k=K, sg_n=sg_n)
    U, Ui = pl.pallas_call(
        body,
        grid=(C, sg_n),
        in_specs=[
            pl.BlockSpec((K, N), lambda i, j: (0, 0)),
            pl.BlockSpec((K, N), lambda i, j: (0, 0)),
            pl.BlockSpec((C, N), lambda i, j: (0, 0)),
            pl.BlockSpec((1, N), lambda i, j: (0, 0)),
            pl.BlockSpec((1, N), lambda i, j: (0, 0)),
            pl.BlockSpec((1, C // sg_n, K, N), lambda i, j: (i, j, 0, 0)),
        ],
        out_specs=[
            pl.BlockSpec((1, 1), lambda i, j: (0, 0)),
            pl.BlockSpec((C, N), lambda i, j: (0, 0)),
        ],
        out_shape=[
            jax.ShapeDtypeStruct((1, 1), jnp.float32),
            jax.ShapeDtypeStruct((C, N), jnp.float32),
        ],
        scratch_shapes=[pltpu.VMEM((C, K, N), jnp.float32)],
    )(sj, mij_kn, h_cn, mask_i, S, Jt)
    return (U.reshape(1), jnp.transpose(Ui, (1, 0))[None])


# speedup vs baseline: 1.1054x; 1.1054x over previous
import functools

import jax
import jax.numpy as jnp
from jax import lax
from jax.experimental import pallas as pl
from jax.experimental.pallas import tpu as pltpu
from jax.experimental.pallas import tpu_sc as plsc

_NC = 2
_NS = 16
_NW = _NC * _NS


_CHUNK = 128
_FIRE = 8


def _sc_gather_body(S_hbm, edge_hbm, out_hbm, S_sh, idx_v, out_v, sem, *, e_per_w, rows):
    sid = lax.axis_index("s")
    wid = sid * _NC + lax.axis_index("c")
    base = wid * e_per_w

    @pl.when(sid == 0)
    def _():
        pltpu.sync_copy(S_hbm, S_sh)

    zero = jnp.zeros((16,), jnp.int32)
    for t in range(e_per_w - e_per_w % 16, rows * _CHUNK, 16):
        idx_v[pl.ds(t, 16)] = zero
    cp_idx = pltpu.make_async_copy(
        edge_hbm.at[pl.ds(base, e_per_w)], idx_v.at[pl.ds(0, e_per_w)], sem)
    cp_idx.start()
    plsc.subcore_barrier()
    cp_idx.wait()

    def fire(j, _):
        pltpu.make_async_copy(
            S_sh.at[idx_v.at[pl.ds(j * _CHUNK, _CHUNK)]],
            out_v.at[pl.ds(j * _CHUNK, _CHUNK)],
            sem,
        ).start()
        return 0

    def drain(j, _):
        pltpu.make_async_copy(
            S_sh.at[idx_v.at[pl.ds(j * _CHUNK, _CHUNK)]],
            out_v.at[pl.ds(j * _CHUNK, _CHUNK)],
            sem,
        ).wait()
        return 0

    lax.fori_loop(0, rows, fire, 0)
    lax.fori_loop(0, rows, drain, 0)
    pltpu.sync_copy(out_v.at[pl.ds(0, e_per_w)],
                    out_hbm.at[pl.ds(base, e_per_w)])


def _neighbor_states(S_flat, edge_flat):
    e = edge_flat.shape[0]
    e_per_w = e // _NW
    rows = (e_per_w + _CHUNK - 1) // _CHUNK
    mesh = plsc.VectorSubcoreMesh(core_axis_name="c", subcore_axis_name="s")
    body = functools.partial(_sc_gather_body, e_per_w=e_per_w, rows=rows)
    return pl.kernel(
        body,
        mesh=mesh,
        out_type=jax.ShapeDtypeStruct((e,), jnp.int32),
        scratch_types=[
            pltpu.VMEM_SHARED((S_flat.shape[0],), jnp.int32),
            pltpu.VMEM((rows * _CHUNK,), jnp.int32),
            pltpu.VMEM((rows * _CHUNK,), jnp.int32),
            pltpu.SemaphoreType.DMA,
        ],
    )(S_flat, edge_flat)


def _tc_body(sj_ref, mij_ref, h_ref, mi_ref, S_ref, J_ref, U_ref, Ui_ref, M_ref, *, c, k):
    ci = pl.program_id(0)

    @pl.when(ci == 0)
    def _():
        sj = sj_ref[...]
        mij = mij_ref[...]
        for s in range(c):
            M_ref[s] = (sj == s).astype(jnp.float32) * mij

    part = J_ref[0, 0] * M_ref[0]
    for s in range(1, c):
        part += J_ref[0, s] * M_ref[s]
    ji = part.sum(axis=0, keepdims=True)
    hm = h_ref[pl.ds(ci, 1), :] * mi_ref[...]
    Ui_ref[pl.ds(ci, 1), :] = hm + ji
    sel = (S_ref[...] == ci).astype(jnp.float32)
    contrib = ((hm + 0.5 * ji) * sel).sum().reshape(1, 1)

    @pl.when(ci == 0)
    def _():
        U_ref[...] = jnp.zeros((1, 1), jnp.float32)

    U_ref[...] += contrib


def kernel(S, h, J, edge_idx, mask_i, mask_ij):
    B, N, K, C, _ = J.shape
    assert B == 1

    S_flat = S[0]
    edge_flat = jnp.transpose(edge_idx[0], (1, 0)).reshape(-1)
    sj = _neighbor_states(S_flat, edge_flat).reshape(K, N)

    Jt = jnp.transpose(J[0], (2, 3, 1, 0))
    h_cn = jnp.transpose(h[0], (1, 0))
    mij_kn = jnp.transpose(mask_ij[0], (1, 0))

    body = functools.partial(_tc_body, c=C, k=K)
    U, Ui = pl.pallas_call(
        body,
        grid=(C,),
        in_specs=[
            pl.BlockSpec((K, N), lambda i: (0, 0)),
            pl.BlockSpec((K, N), lambda i: (0, 0)),
            pl.BlockSpec((C, N), lambda i: (0, 0)),
            pl.BlockSpec((1, N), lambda i: (0, 0)),
            pl.BlockSpec((1, N), lambda i: (0, 0)),
            pl.BlockSpec((1, C, K, N), lambda i: (i, 0, 0, 0)),
        ],
        out_specs=[
            pl.BlockSpec((1, 1), lambda i: (0, 0)),
            pl.BlockSpec((C, N), lambda i: (0, 0)),
        ],
        out_shape=[
            jax.ShapeDtypeStruct((1, 1), jnp.float32),
            jax.ShapeDtypeStruct((C, N), jnp.float32),
        ],
        scratch_shapes=[pltpu.VMEM((C, K, N), jnp.float32)],
    )(sj, mij_kn, h_cn, mask_i, S, Jt)
    return (U.reshape(1), jnp.transpose(Ui, (1, 0))[None])


# speedup vs baseline: 1.1056x; 1.0002x over previous
import functools

import jax
import jax.numpy as jnp
from jax import lax
from jax.experimental import pallas as pl
from jax.experimental.pallas import tpu as pltpu
from jax.experimental.pallas import tpu_sc as plsc

_NC = 2
_NS = 16
_NW = _NC * _NS


_CHUNK = 128


def _sc_gather_body(S_hbm, edge_hbm, out_hbm, S_sh, idx_v, out_v, sem, *, e_per_w, rows):
    sid = lax.axis_index("s")
    wid = sid * _NC + lax.axis_index("c")
    base = wid * e_per_w

    @pl.when(sid == 0)
    def _():
        pltpu.sync_copy(S_hbm, S_sh)

    zero = jnp.zeros((16,), jnp.int32)
    for t in range(e_per_w - e_per_w % 16, rows * _CHUNK, 16):
        idx_v[pl.ds(t, 16)] = zero
    cp_idx = pltpu.make_async_copy(
        edge_hbm.at[pl.ds(base, e_per_w)], idx_v.at[pl.ds(0, e_per_w)], sem)
    cp_idx.start()
    plsc.subcore_barrier()
    cp_idx.wait()

    def fire(j, _):
        pltpu.make_async_copy(
            S_sh.at[idx_v.at[pl.ds(j * _CHUNK, _CHUNK)]],
            out_v.at[pl.ds(j * _CHUNK, _CHUNK)],
            sem,
        ).start()
        return 0

    def drain(j, _):
        pltpu.make_async_copy(
            S_sh.at[idx_v.at[pl.ds(j * _CHUNK, _CHUNK)]],
            out_v.at[pl.ds(j * _CHUNK, _CHUNK)],
            sem,
        ).wait()
        return 0

    lax.fori_loop(0, rows, fire, 0)
    lax.fori_loop(0, rows, drain, 0)
    pltpu.sync_copy(out_v.at[pl.ds(0, e_per_w)],
                    out_hbm.at[pl.ds(base, e_per_w)])


def _neighbor_states(S_flat, edge_flat):
    e = edge_flat.shape[0]
    e_per_w = e // _NW
    rows = (e_per_w + _CHUNK - 1) // _CHUNK
    mesh = plsc.VectorSubcoreMesh(core_axis_name="c", subcore_axis_name="s")
    body = functools.partial(_sc_gather_body, e_per_w=e_per_w, rows=rows)
    return pl.kernel(
        body,
        mesh=mesh,
        out_type=jax.ShapeDtypeStruct((e,), jnp.int32),
        scratch_types=[
            pltpu.VMEM_SHARED((S_flat.shape[0],), jnp.int32),
            pltpu.VMEM((rows * _CHUNK,), jnp.int32),
            pltpu.VMEM((rows * _CHUNK,), jnp.int32),
            pltpu.SemaphoreType.DMA,
        ],
    )(S_flat, edge_flat)


def _tc_body(sj_ref, mij_ref, h_ref, mi_ref, S_ref, J_ref, U_ref, Ui_ref, M_ref, *, c):
    ci = pl.program_id(0)

    @pl.when(ci == 0)
    def _():
        sj = sj_ref[...]
        mij = mij_ref[...]
        for s in range(c):
            M_ref[s] = (sj == s).astype(jnp.float32) * mij

    part = J_ref[0, 0] * M_ref[0]
    for s in range(1, c):
        part += J_ref[0, s] * M_ref[s]
    ji = part.sum(axis=0, keepdims=True)
    hm = h_ref[pl.ds(ci, 1), :] * mi_ref[...]
    Ui_ref[pl.ds(ci, 1), :] = hm + ji
    sel = (S_ref[...] == ci).astype(jnp.float32)
    contrib = ((hm + 0.5 * ji) * sel).sum().reshape(1, 1)

    @pl.when(ci == 0)
    def _():
        U_ref[...] = jnp.zeros((1, 1), jnp.float32)

    U_ref[...] += contrib


def kernel(S, h, J, edge_idx, mask_i, mask_ij):
    B, N, K, C, _ = J.shape
    assert B == 1

    S_flat = S[0]
    edge_flat = jnp.transpose(edge_idx[0], (1, 0)).reshape(-1)
    sj = _neighbor_states(S_flat, edge_flat).reshape(K, N)

    Jt = jnp.transpose(J[0], (2, 3, 1, 0))
    h_cn = jnp.transpose(h[0], (1, 0))
    mij_kn = jnp.transpose(mask_ij[0], (1, 0))

    body = functools.partial(_tc_body, c=C)
    U, Ui = pl.pallas_call(
        body,
        grid=(C,),
        in_specs=[
            pl.BlockSpec((K, N), lambda i: (0, 0)),
            pl.BlockSpec((K, N), lambda i: (0, 0)),
            pl.BlockSpec((C, N), lambda i: (0, 0)),
            pl.BlockSpec((1, N), lambda i: (0, 0)),
            pl.BlockSpec((1, N), lambda i: (0, 0)),
            pl.BlockSpec((1, C, K, N), lambda i: (i, 0, 0, 0)),
        ],
        out_specs=[
            pl.BlockSpec((1, 1), lambda i: (0, 0)),
            pl.BlockSpec((C, N), lambda i: (0, 0)),
        ],
        out_shape=[
            jax.ShapeDtypeStruct((1, 1), jnp.float32),
            jax.ShapeDtypeStruct((C, N), jnp.float32),
        ],
        scratch_shapes=[pltpu.VMEM((C, K, N), jnp.float32)],
    )(sj, mij_kn, h_cn, mask_i, S, Jt)
    return (U.reshape(1), jnp.transpose(Ui, (1, 0))[None])


# speedup vs baseline: 1.1091x; 1.0032x over previous
import functools

import jax
import jax.numpy as jnp
from jax import lax
from jax.experimental import pallas as pl
from jax.experimental.pallas import tpu as pltpu
from jax.experimental.pallas import tpu_sc as plsc

_NC = 2
_NS = 16
_NW = _NC * _NS


_CHUNK = 128


def _sc_gather_body(S_hbm, edge_hbm, out_hbm, S_sh, idx_v, out_v, sem, *, e_per_w, rows):
    sid = lax.axis_index("s")
    wid = sid * _NC + lax.axis_index("c")
    base = wid * e_per_w

    @pl.when(sid == 0)
    def _():
        pltpu.sync_copy(S_hbm, S_sh)

    zero = jnp.zeros((16,), jnp.int32)
    for t in range(e_per_w - e_per_w % 16, rows * _CHUNK, 16):
        idx_v[pl.ds(t, 16)] = zero
    cp_idx = pltpu.make_async_copy(
        edge_hbm.at[pl.ds(base, e_per_w)], idx_v.at[pl.ds(0, e_per_w)], sem)
    cp_idx.start()
    plsc.subcore_barrier()
    cp_idx.wait()

    def fire(j, _):
        pltpu.make_async_copy(
            S_sh.at[idx_v.at[pl.ds(j * _CHUNK, _CHUNK)]],
            out_v.at[pl.ds(j * _CHUNK, _CHUNK)],
            sem,
        ).start()
        return 0

    def drain(j, _):
        pltpu.make_async_copy(
            S_sh.at[idx_v.at[pl.ds(j * _CHUNK, _CHUNK)]],
            out_v.at[pl.ds(j * _CHUNK, _CHUNK)],
            sem,
        ).wait()
        return 0

    lax.fori_loop(0, rows, fire, 0)
    lax.fori_loop(0, rows, drain, 0)
    pltpu.sync_copy(out_v.at[pl.ds(0, e_per_w)],
                    out_hbm.at[pl.ds(base, e_per_w)])


def _neighbor_states(S_flat, edge_flat):
    e = edge_flat.shape[0]
    e_per_w = e // _NW
    rows = (e_per_w + _CHUNK - 1) // _CHUNK
    mesh = plsc.VectorSubcoreMesh(core_axis_name="c", subcore_axis_name="s")
    body = functools.partial(_sc_gather_body, e_per_w=e_per_w, rows=rows)
    return pl.kernel(
        body,
        mesh=mesh,
        out_type=jax.ShapeDtypeStruct((e,), jnp.int32),
        scratch_types=[
            pltpu.VMEM_SHARED((S_flat.shape[0],), jnp.int32),
            pltpu.VMEM((rows * _CHUNK,), jnp.int32),
            pltpu.VMEM((rows * _CHUNK,), jnp.int32),
            pltpu.SemaphoreType.DMA,
        ],
    )(S_flat, edge_flat)


def _tc_body(sj_ref, mij_ref, h_ref, mi_ref, S_ref, J_ref, U_ref, Ui_ref, M_ref, *, c):
    ci = pl.program_id(0)

    @pl.when(ci == 0)
    def _():
        sj = sj_ref[...]
        mij = mij_ref[...]
        for s in range(c):
            M_ref[s] = (sj == s).astype(jnp.float32) * mij

    def halves(s):
        return (J_ref[0, s, 0:8, :] * M_ref[s][0:8, :]
                + J_ref[0, s, 8:16, :] * M_ref[s][8:16, :])

    part = halves(0)
    for s in range(1, c):
        part += halves(s)
    ji = part.sum(axis=0, keepdims=True)
    hm = h_ref[pl.ds(ci, 1), :] * mi_ref[...]
    Ui_ref[pl.ds(ci, 1), :] = hm + ji
    sel = (S_ref[...] == ci).astype(jnp.float32)
    contrib = ((hm + 0.5 * ji) * sel).sum().reshape(1, 1)

    @pl.when(ci == 0)
    def _():
        U_ref[...] = jnp.zeros((1, 1), jnp.float32)

    U_ref[...] += contrib


def kernel(S, h, J, edge_idx, mask_i, mask_ij):
    B, N, K, C, _ = J.shape
    assert B == 1

    S_flat = S[0]
    edge_flat = jnp.transpose(edge_idx[0], (1, 0)).reshape(-1)
    sj = _neighbor_states(S_flat, edge_flat).reshape(K, N)

    Jt = jnp.transpose(J[0], (2, 3, 1, 0))
    h_cn = jnp.transpose(h[0], (1, 0))
    mij_kn = jnp.transpose(mask_ij[0], (1, 0))

    body = functools.partial(_tc_body, c=C)
    U, Ui = pl.pallas_call(
        body,
        grid=(C,),
        in_specs=[
            pl.BlockSpec((K, N), lambda i: (0, 0)),
            pl.BlockSpec((K, N), lambda i: (0, 0)),
            pl.BlockSpec((C, N), lambda i: (0, 0)),
            pl.BlockSpec((1, N), lambda i: (0, 0)),
            pl.BlockSpec((1, N), lambda i: (0, 0)),
            pl.BlockSpec((1, C, K, N), lambda i: (i, 0, 0, 0)),
        ],
        out_specs=[
            pl.BlockSpec((1, 1), lambda i: (0, 0)),
            pl.BlockSpec((C, N), lambda i: (0, 0)),
        ],
        out_shape=[
            jax.ShapeDtypeStruct((1, 1), jnp.float32),
            jax.ShapeDtypeStruct((C, N), jnp.float32),
        ],
        scratch_shapes=[pltpu.VMEM((C, K, N), jnp.float32)],
    )(sj, mij_kn, h_cn, mask_i, S, Jt)
    return (U.reshape(1), jnp.transpose(Ui, (1, 0))[None])
